# baseline (speedup 1.0000x reference)
import dataclasses
import functools

import jax
import jax.numpy as jnp
from jax.experimental import pallas as pl
from jax.experimental.pallas import tpu as pltpu
from jax.experimental.pallas import tpu_sc as plsc

_CS = 512
_LANES = 16

_INT_MIN = -(2 ** 31)


def _score_key_kernel(am_ref, h_ref, w_ref, b_ref, out_ref, scores_scr, *, nch, cs):
    j = pl.program_id(1)

    h = h_ref[0]
    w = w_ref[...]
    scores = jax.lax.dot_general(
        h.astype(jnp.bfloat16),
        w.reshape(-1, 1).astype(jnp.bfloat16),
        (((1,), (0,)), ((), ())),
        preferred_element_type=jnp.float32,
    )
    scores_scr[pl.ds(j, 1), :] = scores.reshape(1, cs)

    @pl.when(j == nch - 1)
    def _():
        am = am_ref[0] != 0
        s = scores_scr[...] + b_ref[0, 0]
        bits = jax.lax.bitcast_convert_type(s, jnp.int32)
        skey = jnp.where(bits >= 0, bits, bits ^ jnp.int32(0x7FFFFFFF))
        out_ref[0] = jnp.where(am, skey, jnp.int32(_INT_MIN))


def _compute_score_keys(hidden_states, active_mask, W, b):
    B, S, D = hidden_states.shape
    cs = _CS
    nch = S // cs

    am3 = active_mask.reshape(B, nch, cs).astype(jnp.int32)
    w2 = W.reshape(1, D)
    b2 = b.reshape(1, 1)

    return pl.pallas_call(
        functools.partial(_score_key_kernel, nch=nch, cs=cs),
        grid=(B, nch),
        in_specs=[
            pl.BlockSpec((1, nch, cs), lambda i, j: (i, 0, 0)),
            pl.BlockSpec((1, cs, D), lambda i, j: (i, j, 0)),
            pl.BlockSpec((1, D), lambda i, j: (0, 0)),
            pl.BlockSpec((1, 1), lambda i, j: (0, 0)),
        ],
        out_specs=pl.BlockSpec((1, nch, cs), lambda i, j: (i, 0, 0)),
        out_shape=jax.ShapeDtypeStruct((B, nch, cs), jnp.int32),
        scratch_shapes=[pltpu.VMEM((nch, cs), jnp.float32)],
    )(am3, hidden_states, w2, b2)


def _sc_select(skey, B, S):
    nv = S // _LANES

    mesh = plsc.VectorSubcoreMesh(core_axis_name="c", subcore_axis_name="s",
                                  num_cores=2)
    cp = pltpu.CompilerParams()
    if "needs_layout_passes" in pltpu.CompilerParams.__dataclass_fields__:
        cp = dataclasses.replace(cp, needs_layout_passes=False)

    @functools.partial(
        pl.kernel,
        out_type=jax.ShapeDtypeStruct((B, nv, _LANES), jnp.int32),
        mesh=mesh,
        compiler_params=cp,
        scratch_types=[
            pltpu.VMEM((nv, _LANES), jnp.int32),
            pltpu.VMEM((nv, _LANES), jnp.int32),
            pltpu.SemaphoreType.DMA,
        ],
    )
    def select(skey_hbm, out_hbm, keys, outm, sem):
        cid = jax.lax.axis_index("c")
        sid = jax.lax.axis_index("s")
        row = cid * (B // 2) + sid

        @pl.when(sid < B // 2)
        def _():
            pltpu.async_copy(skey_hbm.at[row], keys, sem).wait()

            zeros = jnp.zeros((_LANES,), jnp.int32)
            ones = jnp.ones((_LANES,), jnp.int32)
            U = 8

            def count_ge3(t1, t2, t3):
                def body(i, accs):
                    a1, a2, a3 = accs
                    for u in range(U):
                        kv = keys[i * U + u]
                        a1 = a1 + jnp.where(kv >= t1, ones, zeros)
                        a2 = a2 + jnp.where(kv >= t2, ones, zeros)
                        a3 = a3 + jnp.where(kv >= t3, ones, zeros)
                    return (a1, a2, a3)
                a1, a2, a3 = jax.lax.fori_loop(0, nv // U, body,
                                               (zeros, zeros, zeros))
                return jnp.sum(a1), jnp.sum(a2), jnp.sum(a3)

            a_, n_pos, _ = count_ge3(jnp.int32(_INT_MIN + 1), jnp.int32(0),
                                     jnp.int32(0))
            a = a_
            k = (a + 1) // 2

            v0 = jnp.where(n_pos >= k, jnp.int32(0), jnp.int32(_INT_MIN))

            def vbody(j, v):
                bit = jnp.left_shift(jnp.int32(1), jnp.int32(29) - 2 * j)
                c1, c2, c3 = count_ge3(v + bit, v + 2 * bit, v + 3 * bit)
                inc = jnp.where(
                    c3 >= k, jnp.int32(3),
                    jnp.where(c2 >= k, jnp.int32(2),
                              jnp.where(c1 >= k, jnp.int32(1), jnp.int32(0))))
                return v + inc * bit

            v = jax.lax.fori_loop(0, 15, vbody, v0)

            c_v, c_v1, c_v2 = count_ge3(v, v + 1, v + 2)
            take = c_v1 >= k
            v = jnp.where(take, v + 1, v)
            n_ge = jnp.where(take, c_v1, c_v)
            n_gt = jnp.where(take, c_v2, c_v1)

            e = n_ge - n_gt
            t = k - n_gt
            lane = jax.lax.iota(jnp.int32, _LANES)

            def count_eq_below(m):
                def body(i, accs):
                    out = []
                    for u in range(U):
                        idx = lane + (i * U + u) * _LANES
                        hit = (keys[i * U + u] == v) & (idx < m)
                        out.append(accs[u] + jnp.where(hit, ones, zeros))
                    return tuple(out)
                accs = jax.lax.fori_loop(0, nv // U, body, (zeros,) * U)
                return jnp.sum(sum(accs[1:], accs[0]))

            def mbody(i, m):
                bit = jnp.left_shift(jnp.int32(1), jnp.int32(11) - i)
                trial = m + bit
                return jnp.where(count_eq_below(trial) < t, trial, m)

            m = jax.lax.cond(
                t == e,
                lambda: jnp.int32(S - 1),
                lambda: jax.lax.fori_loop(0, 12, mbody, jnp.int32(0)),
            )

            keep_any = k > 0

            def wbody(i, _):
                for u in range(U):
                    idx = lane + (i * U + u) * _LANES
                    kv = keys[i * U + u]
                    keep = (kv > v) | ((kv == v) & (idx <= m))
                    keep = keep & keep_any
                    outm[i * U + u] = jnp.where(keep, ones, zeros)
                return 0

            jax.lax.fori_loop(0, nv // U, wbody, 0)

            pltpu.async_copy(outm, out_hbm.at[row], sem).wait()

    return select(skey)


def kernel(hidden_states, active_mask, W, b):
    B, S, D = hidden_states.shape
    skey = _compute_score_keys(hidden_states, active_mask, W, b)
    skey = skey.reshape(B, S // _LANES, _LANES)
    mask = _sc_select(skey, B, S)
    return mask.reshape(B, S).astype(jnp.bool_)


# speedup vs baseline: 1.0492x; 1.0492x over previous
import dataclasses
import functools

import jax
import jax.numpy as jnp
from jax.experimental import pallas as pl
from jax.experimental.pallas import tpu as pltpu
from jax.experimental.pallas import tpu_sc as plsc

_CS = 512
_LANES = 16

_INT_MIN = -(2 ** 31)


def _score_key_kernel(am_ref, h_ref, w_ref, b_ref, out_ref, scores_scr, *, nch, cs):
    j = pl.program_id(1)

    h = h_ref[0]
    w = w_ref[...]
    scores = jax.lax.dot_general(
        h.astype(jnp.bfloat16),
        w.reshape(-1, 1).astype(jnp.bfloat16),
        (((1,), (0,)), ((), ())),
        preferred_element_type=jnp.float32,
    )
    scores_scr[pl.ds(j, 1), :] = scores.reshape(1, cs)

    @pl.when(j == nch - 1)
    def _():
        am = am_ref[0] != 0
        s = scores_scr[...] + b_ref[0, 0]
        bits = jax.lax.bitcast_convert_type(s, jnp.int32)
        skey = jnp.where(bits >= 0, bits, bits ^ jnp.int32(0x7FFFFFFF))
        out_ref[0] = jnp.where(am, skey, jnp.int32(_INT_MIN))


def _compute_score_keys(hidden_states, active_mask, W, b):
    B, S, D = hidden_states.shape
    cs = _CS
    nch = S // cs

    am3 = active_mask.reshape(B, nch, cs).astype(jnp.int32)
    w2 = W.reshape(1, D)
    b2 = b.reshape(1, 1)

    return pl.pallas_call(
        functools.partial(_score_key_kernel, nch=nch, cs=cs),
        grid=(B, nch),
        in_specs=[
            pl.BlockSpec((1, nch, cs), lambda i, j: (i, 0, 0)),
            pl.BlockSpec((1, cs, D), lambda i, j: (i, j, 0)),
            pl.BlockSpec((1, D), lambda i, j: (0, 0)),
            pl.BlockSpec((1, 1), lambda i, j: (0, 0)),
        ],
        out_specs=pl.BlockSpec((1, nch, cs), lambda i, j: (i, 0, 0)),
        out_shape=jax.ShapeDtypeStruct((B, nch, cs), jnp.int32),
        scratch_shapes=[pltpu.VMEM((nch, cs), jnp.float32)],
    )(am3, hidden_states, w2, b2)


def _sc_select(skey, B, S):
    nv = S // _LANES

    mesh = plsc.VectorSubcoreMesh(core_axis_name="c", subcore_axis_name="s",
                                  num_cores=2)
    cp = pltpu.CompilerParams()
    if "needs_layout_passes" in pltpu.CompilerParams.__dataclass_fields__:
        cp = dataclasses.replace(cp, needs_layout_passes=False)

    @functools.partial(
        pl.kernel,
        out_type=jax.ShapeDtypeStruct((B, nv, _LANES), jnp.int32),
        mesh=mesh,
        compiler_params=cp,
        scratch_types=[
            pltpu.VMEM((nv, _LANES), jnp.int32),
            pltpu.VMEM((nv, _LANES), jnp.int32),
            pltpu.SemaphoreType.DMA,
        ],
    )
    def select(skey_hbm, out_hbm, keys, outm, sem):
        cid = jax.lax.axis_index("c")
        sid = jax.lax.axis_index("s")
        row = cid * (B // 2) + sid

        @pl.when(sid < B // 2)
        def _():
            pltpu.async_copy(skey_hbm.at[row], keys, sem).wait()

            zeros = jnp.zeros((_LANES,), jnp.int32)
            ones = jnp.ones((_LANES,), jnp.int32)
            U = 4

            def count_ge3(t1, t2, t3):
                def body(i, accs):
                    a1, a2, a3 = accs
                    for u in range(U):
                        kv = keys[i * U + u]
                        a1 = a1 + jnp.where(kv >= t1, ones, zeros)
                        a2 = a2 + jnp.where(kv >= t2, ones, zeros)
                        a3 = a3 + jnp.where(kv >= t3, ones, zeros)
                    return (a1, a2, a3)
                a1, a2, a3 = jax.lax.fori_loop(0, nv // U, body,
                                               (zeros, zeros, zeros))
                return jnp.sum(a1), jnp.sum(a2), jnp.sum(a3)

            a_, n_pos, _ = count_ge3(jnp.int32(_INT_MIN + 1), jnp.int32(0),
                                     jnp.int32(0))
            a = a_
            k = (a + 1) // 2

            v0 = jnp.where(n_pos >= k, jnp.int32(0), jnp.int32(_INT_MIN))

            def vbody(j, v):
                bit = jnp.left_shift(jnp.int32(1), jnp.int32(29) - 2 * j)
                c1, c2, c3 = count_ge3(v + bit, v + 2 * bit, v + 3 * bit)
                inc = jnp.where(
                    c3 >= k, jnp.int32(3),
                    jnp.where(c2 >= k, jnp.int32(2),
                              jnp.where(c1 >= k, jnp.int32(1), jnp.int32(0))))
                return v + inc * bit

            v = jax.lax.fori_loop(0, 15, vbody, v0)

            c_v, c_v1, c_v2 = count_ge3(v, v + 1, v + 2)
            take = c_v1 >= k
            v = jnp.where(take, v + 1, v)
            n_ge = jnp.where(take, c_v1, c_v)
            n_gt = jnp.where(take, c_v2, c_v1)

            e = n_ge - n_gt
            t = k - n_gt
            lane = jax.lax.iota(jnp.int32, _LANES)

            def count_eq_below(m):
                def body(i, accs):
                    out = []
                    for u in range(U):
                        idx = lane + (i * U + u) * _LANES
                        hit = (keys[i * U + u] == v) & (idx < m)
                        out.append(accs[u] + jnp.where(hit, ones, zeros))
                    return tuple(out)
                accs = jax.lax.fori_loop(0, nv // U, body, (zeros,) * U)
                return jnp.sum(sum(accs[1:], accs[0]))

            def mbody(i, m):
                bit = jnp.left_shift(jnp.int32(1), jnp.int32(11) - i)
                trial = m + bit
                return jnp.where(count_eq_below(trial) < t, trial, m)

            m = jax.lax.cond(
                t == e,
                lambda: jnp.int32(S - 1),
                lambda: jax.lax.fori_loop(0, 12, mbody, jnp.int32(0)),
            )

            keep_any = k > 0

            def wbody(i, _):
                for u in range(U):
                    idx = lane + (i * U + u) * _LANES
                    kv = keys[i * U + u]
                    keep = (kv > v) | ((kv == v) & (idx <= m))
                    keep = keep & keep_any
                    outm[i * U + u] = jnp.where(keep, ones, zeros)
                return 0

            jax.lax.fori_loop(0, nv // U, wbody, 0)

            pltpu.async_copy(outm, out_hbm.at[row], sem).wait()

    return select(skey)


def kernel(hidden_states, active_mask, W, b):
    B, S, D = hidden_states.shape
    skey = _compute_score_keys(hidden_states, active_mask, W, b)
    skey = skey.reshape(B, S // _LANES, _LANES)
    mask = _sc_select(skey, B, S)
    return mask.reshape(B, S).astype(jnp.bool_)
